# baseline (speedup 1.0000x reference)
import jax
import jax.numpy as jnp
from jax.experimental import pallas as pl


def _body(f1_ref, f2_ref, w_ref, o_ref):
    f1t = f1_ref[...].astype(jnp.bfloat16).T
    f2t = f2_ref[...].astype(jnp.bfloat16).T
    n1, bz = f1t.shape
    n2 = f2t.shape[0]
    big = (f1t[:, None, :] * f2t[None, :, :]).reshape(n1 * n2, bz)
    o_ref[...] = jnp.dot(w_ref[...], big, preferred_element_type=jnp.float32).T


def kernel(features_1, features_2, mixing_matrix):
    z, n1 = features_1.shape
    n2 = features_2.shape[1]
    n_out = mixing_matrix.shape[0]
    w = mixing_matrix.astype(jnp.bfloat16)
    bz = 512
    return pl.pallas_call(
        _body,
        grid=(z // bz,),
        in_specs=[
            pl.BlockSpec((bz, n1), lambda g: (g, 0)),
            pl.BlockSpec((bz, n2), lambda g: (g, 0)),
            pl.BlockSpec((n_out, n1 * n2), lambda g: (0, 0)),
        ],
        out_specs=pl.BlockSpec((bz, n_out), lambda g: (g, 0)),
        out_shape=jax.ShapeDtypeStruct((z, n_out), jnp.float32),
    )(features_1, features_2, w)


# speedup vs baseline: 1.2170x; 1.2170x over previous
import jax
import jax.numpy as jnp
from jax.experimental import pallas as pl


def _body(f1_ref, f2_ref, w_ref, o_ref):
    f1t = f1_ref[...].astype(jnp.bfloat16).T
    f2t = f2_ref[...].astype(jnp.bfloat16).T
    n1, bz = f1t.shape
    n2 = f2t.shape[0]
    big = (f1t[:, None, :] * f2t[None, :, :]).reshape(n1 * n2, bz)
    o_ref[...] = jnp.dot(w_ref[...], big, preferred_element_type=jnp.float32)


def kernel(features_1, features_2, mixing_matrix):
    z, n1 = features_1.shape
    n2 = features_2.shape[1]
    n_out = mixing_matrix.shape[0]
    w = mixing_matrix.astype(jnp.bfloat16)
    bz = 512
    outt = pl.pallas_call(
        _body,
        grid=(z // bz,),
        in_specs=[
            pl.BlockSpec((bz, n1), lambda g: (g, 0)),
            pl.BlockSpec((bz, n2), lambda g: (g, 0)),
            pl.BlockSpec((n_out, n1 * n2), lambda g: (0, 0)),
        ],
        out_specs=pl.BlockSpec((n_out, bz), lambda g: (0, g)),
        out_shape=jax.ShapeDtypeStruct((n_out, z), jnp.float32),
    )(features_1, features_2, w)
    return outt.T


# speedup vs baseline: 1.4593x; 1.1991x over previous
import jax
import jax.numpy as jnp
from jax.experimental import pallas as pl


def _body(f1_ref, f2_ref, w_ref, o_ref):
    f1t = f1_ref[...].astype(jnp.bfloat16).T
    f2t = f2_ref[...].astype(jnp.bfloat16).T
    n1, bz = f1t.shape
    n2 = f2t.shape[0]
    big = (f1t[:, None, :] * f2t[None, :, :]).reshape(n1 * n2, bz)
    w = w_ref[...].astype(jnp.bfloat16)
    o_ref[...] = jnp.dot(w, big, preferred_element_type=jnp.float32)


def kernel(features_1, features_2, mixing_matrix):
    z, n1 = features_1.shape
    n2 = features_2.shape[1]
    n_out = mixing_matrix.shape[0]
    bz = 1024
    outt = pl.pallas_call(
        _body,
        grid=(z // bz,),
        in_specs=[
            pl.BlockSpec((bz, n1), lambda g: (g, 0)),
            pl.BlockSpec((bz, n2), lambda g: (g, 0)),
            pl.BlockSpec((n_out, n1 * n2), lambda g: (0, 0)),
        ],
        out_specs=pl.BlockSpec((n_out, bz), lambda g: (0, g)),
        out_shape=jax.ShapeDtypeStruct((n_out, z), jnp.float32),
    )(features_1, features_2, mixing_matrix)
    return outt.T
